# baseline (speedup 1.0000x reference)
import functools

import jax
import jax.numpy as jnp
from jax import lax
from jax.experimental import pallas as pl
from jax.experimental.pallas import tpu as pltpu
from jax.experimental.pallas import tpu_sc as plsc

B, L, D = 16384, 20, 128
NC, NS = 2, 16
NW = NC * NS
NPW = B // NW
CH = 16
NCHK = NPW // CH
IDS = CH * L

_mesh = plsc.VectorSubcoreMesh(core_axis_name="c", subcore_axis_name="s")


_GSPLIT = ((0, 128), (128, 128), (256, 64))


@functools.partial(
    pl.kernel,
    mesh=_mesh,
    out_type=[
        jax.ShapeDtypeStruct((B, D), jnp.float32),
        jax.ShapeDtypeStruct((B, D), jnp.float32),
    ],
    scratch_types=[
        pltpu.VMEM((NPW,), jnp.int32),
        pltpu.VMEM((NPW * L,), jnp.int32),
        pltpu.VMEM((4, CH, D), jnp.float32),
        pltpu.VMEM((2, IDS, D), jnp.float32),
        pltpu.VMEM((4, CH, D), jnp.float32),
        pltpu.SemaphoreType.DMA,
        pltpu.SemaphoreType.DMA,
        pltpu.SemaphoreType.DMA,
        pltpu.SemaphoreType.DMA,
        pltpu.SemaphoreType.DMA,
        pltpu.SemaphoreType.DMA,
    ],
)
def _gather_pool(nodes_h, gids_h, user_h, item_h, oself_h, opool_h,
                 nidx, gidx, sbuf, nbuf, pbuf,
                 gsem0, gsem1, wsem0, wsem1, wsem2, wsem3):
    gsem = (gsem0, gsem1)
    wsem = (wsem0, wsem1, wsem2, wsem3)
    wid = lax.axis_index("s") * NC + lax.axis_index("c")
    base = wid * NPW
    pltpu.sync_copy(nodes_h.at[pl.ds(base, NPW)], nidx)
    pltpu.sync_copy(gids_h.at[pl.ds(base * L, NPW * L)], gidx)

    def gather_descs(g, sb, nb):
        c0 = g * CH
        ib = g * IDS
        ds = [pltpu.make_async_copy(user_h.at[nidx.at[pl.ds(c0, CH)]],
                                    sbuf.at[sb], gsem[nb])]
        for off, n in _GSPLIT:
            ds.append(pltpu.make_async_copy(
                item_h.at[gidx.at[pl.ds(ib + off, n)]],
                nbuf.at[nb, pl.ds(off, n)], gsem[nb]))
        return ds

    def write_descs(g, sb):
        c0 = g * CH
        return [
            pltpu.make_async_copy(sbuf.at[sb], oself_h.at[pl.ds(base + c0, CH)],
                                  wsem[sb]),
            pltpu.make_async_copy(pbuf.at[sb], opool_h.at[pl.ds(base + c0, CH)],
                                  wsem[sb]),
        ]

    def issue_g(g, sb, nb):
        for d in gather_descs(g, sb, nb):
            d.start()

    def drain_g(g, sb, nb):
        for d in gather_descs(g, sb, nb):
            d.wait()

    def issue_w(g, sb):
        for d in write_descs(g, sb):
            d.start()

    def drain_w(g, sb):
        for d in write_descs(g, sb):
            d.wait()

    def compute(g, sb, nb):
        def node(i, c2):
            rb = i * L
            for d in range(D // 16):
                sl = pl.ds(d * 16, 16)
                acc = nbuf[nb, rb, sl]
                for j in range(1, L):
                    acc = acc + nbuf[nb, rb + j, sl]
                pbuf[sb, i, sl] = acc * (1.0 / L)
            return c2
        lax.fori_loop(0, CH, node, 0)

    issue_g(0, 0, 0)

    def body(g2, carry):
        for b in range(4):
            g = 4 * g2 + b
            sb = b
            nb = b % 2
            sb1 = (b + 1) % 4
            nb1 = (b + 1) % 2
            if b < 3:
                @pl.when(g2 >= 1)
                def _():
                    drain_w(g - 3, sb1)
                issue_g(g + 1, sb1, nb1)
            else:
                drain_w(g - 3, sb1)

                @pl.when(g2 < NCHK // 4 - 1)
                def _():
                    issue_g(g + 1, sb1, nb1)
            drain_g(g, sb, nb)
            compute(g, sb, nb)
            issue_w(g, sb)
        return carry

    lax.fori_loop(0, NCHK // 4, body, 0)
    drain_w(NCHK - 3, 1)
    drain_w(NCHK - 2, 2)
    drain_w(NCHK - 1, 3)


TB = 2048


def _mm_body(s_ref, p_ref, w1_ref, w2_ref, b_ref, o_ref):
    acc = jnp.dot(s_ref[...], w1_ref[...], preferred_element_type=jnp.float32)
    acc = acc + jnp.dot(p_ref[...], w2_ref[...],
                        preferred_element_type=jnp.float32)
    o_ref[...] = jnp.maximum(acc + b_ref[...], 0.0)


def _combine(self_f, pool_f, W1, W2, b2):
    return pl.pallas_call(
        _mm_body,
        grid=(B // TB,),
        in_specs=[
            pl.BlockSpec((TB, D), lambda i: (i, 0)),
            pl.BlockSpec((TB, D), lambda i: (i, 0)),
            pl.BlockSpec((D, D), lambda i: (0, 0)),
            pl.BlockSpec((D, D), lambda i: (0, 0)),
            pl.BlockSpec((1, D), lambda i: (0, 0)),
        ],
        out_specs=pl.BlockSpec((TB, D), lambda i: (i, 0)),
        out_shape=jax.ShapeDtypeStruct((B, D), jnp.float32),
    )(self_f, pool_f, W1, W2, b2)


def kernel(nodes, neigh_idx, user_table, item_table, W, b):
    gids = neigh_idx.reshape(-1)
    self_f, pool_f = _gather_pool(nodes, gids, user_table, item_table)
    return _combine(self_f, pool_f, W[:D], W[D:], b.reshape(1, D))


# speedup vs baseline: 1.3667x; 1.3667x over previous
import functools

import jax
import jax.numpy as jnp
from jax import lax
from jax.experimental import pallas as pl
from jax.experimental.pallas import tpu as pltpu
from jax.experimental.pallas import tpu_sc as plsc

B, L, D = 16384, 20, 128
NC, NS = 2, 16
NW = NC * NS
NPW = B // NW
CH = 16
NCHK = NPW // CH
IDS = CH * L

_mesh = plsc.VectorSubcoreMesh(core_axis_name="c", subcore_axis_name="s")


_GSPLIT = ((0, 128), (128, 128), (256, 64))


@functools.partial(
    pl.kernel,
    mesh=_mesh,
    out_type=[
        jax.ShapeDtypeStruct((B, D), jnp.float32),
        jax.ShapeDtypeStruct((B, D), jnp.float32),
    ],
    scratch_types=[
        pltpu.VMEM((NPW,), jnp.int32),
        pltpu.VMEM((NPW * L,), jnp.int32),
        pltpu.VMEM((4, CH, D), jnp.float32),
        pltpu.VMEM((2, IDS, D), jnp.float32),
        pltpu.VMEM((4, CH, D), jnp.float32),
        pltpu.SemaphoreType.DMA,
        pltpu.SemaphoreType.DMA,
        pltpu.SemaphoreType.DMA,
        pltpu.SemaphoreType.DMA,
        pltpu.SemaphoreType.DMA,
        pltpu.SemaphoreType.DMA,
    ],
)
def _gather_pool(nodes_h, gids_h, user_h, item_h, oself_h, opool_h,
                 nidx, gidx, sbuf, nbuf, pbuf,
                 gsem0, gsem1, wsem0, wsem1, wsem2, wsem3):
    gsem = (gsem0, gsem1)
    wsem = (wsem0, wsem1, wsem2, wsem3)
    wid = lax.axis_index("s") * NC + lax.axis_index("c")
    base = wid * NPW
    pltpu.sync_copy(nodes_h.at[pl.ds(base, NPW)], nidx)
    pltpu.sync_copy(gids_h.at[pl.ds(base * L, NPW * L)], gidx)

    def gather_descs(g, sb, nb):
        c0 = g * CH
        ib = g * IDS
        ds = [pltpu.make_async_copy(user_h.at[nidx.at[pl.ds(c0, CH)]],
                                    sbuf.at[sb], gsem[nb])]
        for off, n in _GSPLIT:
            ds.append(pltpu.make_async_copy(
                item_h.at[gidx.at[pl.ds(ib + off, n)]],
                nbuf.at[nb, pl.ds(off, n)], gsem[nb]))
        return ds

    def write_descs(g, sb):
        c0 = g * CH
        return [
            pltpu.make_async_copy(sbuf.at[sb], oself_h.at[pl.ds(base + c0, CH)],
                                  wsem[sb]),
            pltpu.make_async_copy(pbuf.at[sb], opool_h.at[pl.ds(base + c0, CH)],
                                  wsem[sb]),
        ]

    def issue_g(g, sb, nb):
        for d in gather_descs(g, sb, nb):
            d.start()

    def drain_g(g, sb, nb):
        for d in gather_descs(g, sb, nb):
            d.wait()

    def issue_w(g, sb):
        for d in write_descs(g, sb):
            d.start()

    def drain_w(g, sb):
        for d in write_descs(g, sb):
            d.wait()

    def compute(g, sb, nb):
        @plsc.parallel_loop(0, CH, step=1, unroll=2)
        def node(i):
            rb = i * L
            for d in range(D // 16):
                sl = pl.ds(d * 16, 16)
                acc = nbuf[nb, rb, sl]
                for j in range(1, L):
                    acc = acc + nbuf[nb, rb + j, sl]
                pbuf[sb, i, sl] = acc * (1.0 / L)

    issue_g(0, 0, 0)

    def body(g2, carry):
        for b in range(4):
            g = 4 * g2 + b
            sb = b
            nb = b % 2
            sb1 = (b + 1) % 4
            nb1 = (b + 1) % 2
            if b < 3:
                @pl.when(g2 >= 1)
                def _():
                    drain_w(g - 3, sb1)
                issue_g(g + 1, sb1, nb1)
            else:
                drain_w(g - 3, sb1)

                @pl.when(g2 < NCHK // 4 - 1)
                def _():
                    issue_g(g + 1, sb1, nb1)
            drain_g(g, sb, nb)
            compute(g, sb, nb)
            issue_w(g, sb)
        return carry

    lax.fori_loop(0, NCHK // 4, body, 0)
    drain_w(NCHK - 3, 1)
    drain_w(NCHK - 2, 2)
    drain_w(NCHK - 1, 3)


TB = 2048


def _mm_body(s_ref, p_ref, w1_ref, w2_ref, b_ref, o_ref):
    acc = jnp.dot(s_ref[...], w1_ref[...], preferred_element_type=jnp.float32)
    acc = acc + jnp.dot(p_ref[...], w2_ref[...],
                        preferred_element_type=jnp.float32)
    o_ref[...] = jnp.maximum(acc + b_ref[...], 0.0)


def _combine(self_f, pool_f, W1, W2, b2):
    return pl.pallas_call(
        _mm_body,
        grid=(B // TB,),
        in_specs=[
            pl.BlockSpec((TB, D), lambda i: (i, 0)),
            pl.BlockSpec((TB, D), lambda i: (i, 0)),
            pl.BlockSpec((D, D), lambda i: (0, 0)),
            pl.BlockSpec((D, D), lambda i: (0, 0)),
            pl.BlockSpec((1, D), lambda i: (0, 0)),
        ],
        out_specs=pl.BlockSpec((TB, D), lambda i: (i, 0)),
        out_shape=jax.ShapeDtypeStruct((B, D), jnp.float32),
    )(self_f, pool_f, W1, W2, b2)


def kernel(nodes, neigh_idx, user_table, item_table, W, b):
    gids = neigh_idx.reshape(-1)
    self_f, pool_f = _gather_pool(nodes, gids, user_table, item_table)
    return _combine(self_f, pool_f, W[:D], W[D:], b.reshape(1, D))
